# baseline (speedup 1.0000x reference)
import functools

import jax
import jax.numpy as jnp
from jax import lax
from jax.experimental import pallas as pl
from jax.experimental.pallas import tpu as pltpu
from jax.experimental.pallas import tpu_sc as plsc

BATCH = 4096
HIST = 200
HPAD = 208
HC = HPAD // 2
RG = HPAD // 16
DIM = 64
OUT = 256
NCORES = 2
NSUB = 16
NW = NCORES * NSUB
BPW = BATCH // NW
LANES = 16
CG = DIM // LANES
VPAIR = 500000

_mesh = plsc.VectorSubcoreMesh(core_axis_name="c", subcore_axis_name="s")


@functools.partial(
    pl.kernel,
    out_type=jax.ShapeDtypeStruct((BATCH // 2, 2 * DIM), jnp.float32),
    mesh=_mesh,
    compiler_params=pltpu.CompilerParams(use_tc_tiling_on_sc=True),
    scratch_types=[
        pltpu.VMEM((2 * BPW, HC), jnp.int32),
        pltpu.VMEM((BPW, HPAD), jnp.int32),
        pltpu.VMEM((2, HPAD, 2 * DIM), jnp.float32),
        pltpu.VMEM((BPW // 2, 2 * DIM), jnp.float32),
        pltpu.SemaphoreType.DMA,
        pltpu.SemaphoreType.DMA,
    ],
)
def _gather_max(pair_hbm, par_hbm, table_hbm, out_hbm,
                idx_v, par_v, rows_v, pooled_v, sem0, sem1):
    wid = lax.axis_index("s") * NCORES + lax.axis_index("c")
    sems = (sem0, sem1)

    pltpu.sync_copy(pair_hbm.at[pl.ds(wid * 2 * BPW, 2 * BPW)], idx_v)
    pltpu.sync_copy(par_hbm.at[pl.ds(wid * BPW, BPW)], par_v)

    def fire(b, buf):
        for j in range(2):
            pltpu.async_copy(
                table_hbm.at[idx_v.at[2 * b + j]],
                rows_v.at[buf, pl.ds(j * HC, HC)], sems[buf])

    def wait_buf(b, buf):
        for j in range(2):
            pltpu.make_async_copy(
                table_hbm.at[idx_v.at[2 * b + j]],
                rows_v.at[buf, pl.ds(j * HC, HC)], sems[buf]).wait()

    fire(0, 0)
    fire(1, 1)

    @pl.loop(0, BPW, step=2)
    def _pipeline(g):
        for d in range(2):
            b = g + d
            wait_buf(b, d)

            neg = jnp.full((LANES,), -jnp.inf, dtype=jnp.float32)

            def reduce_group(rg, a):
                pvec = par_v[b, pl.ds(rg * LANES, LANES)]
                for l in range(LANES):
                    r = rg * LANES + l
                    p = pvec[l]
                    sel = []
                    for c in range(CG):
                        lo = rows_v[d, r, pl.ds(c * LANES, LANES)]
                        hi = rows_v[d, r, pl.ds(DIM + c * LANES, LANES)]
                        sel.append(jnp.where(p == 1, hi, lo))
                    a = tuple(jnp.maximum(a[c], sel[c]) for c in range(CG))
                return a

            acc = lax.fori_loop(0, RG, reduce_group, (neg, neg, neg, neg))
            for c in range(CG):
                pooled_v[g // 2, pl.ds(d * DIM + c * LANES, LANES)] = acc[c]

            nb = b + 2

            @pl.when(nb < BPW)
            def _():
                fire(nb, d)

    pltpu.sync_copy(pooled_v, out_hbm.at[pl.ds(wid * (BPW // 2), BPW // 2)])


def _linear(pooled, W, b2):
    blk = 512
    grid = BATCH // blk

    def body(p_ref, w_ref, b_ref, o_ref):
        o_ref[...] = lax.dot_general(
            p_ref[...], w_ref[...], (((1,), (1,)), ((), ())),
            preferred_element_type=jnp.float32) + b_ref[...]

    return pl.pallas_call(
        body,
        grid=(grid,),
        in_specs=[
            pl.BlockSpec((blk, DIM), lambda i: (i, 0)),
            pl.BlockSpec((OUT, DIM), lambda i: (0, 0)),
            pl.BlockSpec((1, OUT), lambda i: (0, 0)),
        ],
        out_specs=pl.BlockSpec((blk, OUT), lambda i: (i, 0)),
        out_shape=jax.ShapeDtypeStruct((BATCH, OUT), jnp.float32),
    )(pooled, W, b2)


@jax.jit
def kernel(text, table, W, b):
    text = text.astype(jnp.int32)
    pad = jnp.broadcast_to(text[:, :1], (BATCH, HPAD - HIST))
    text_p = jnp.concatenate([text, pad], axis=1)
    pair = (text_p >> 1).reshape(2 * BATCH, HC)
    par = text_p & 1
    table2 = table.reshape(VPAIR, 2 * DIM)
    pooled2 = _gather_max(pair, par, table2)
    pooled = pooled2.reshape(BATCH, DIM)
    return _linear(pooled, W, b.reshape(1, OUT))
